# baseline (speedup 1.0000x reference)
import functools

import numpy as np

import jax
import jax.numpy as jnp
from jax import lax
from jax.experimental import pallas as pl
from jax.experimental.pallas import tpu as pltpu
from jax.experimental.pallas import tpu_sc as plsc

NC = 2
NS = 16
WIN = 120


PADROWS = 16
NB = 3


@functools.cache
def _make_sc_layer(N, E, D):
    NW = NC * NS
    assert E % NW == 0
    EC = E // NW
    WPT = -(-(-(-EC // WIN)) // NB) * NB
    NA = N + PADROWS
    G = WPT // NB
    RPT = (NA // NS) // 8 * 8
    REM = NA - RPT * NS
    assert REM % 8 == 0
    ORPT = (N // NS) // 8 * 8
    OREM = N - ORPT * NS

    mesh = plsc.VectorSubcoreMesh(core_axis_name="c", subcore_axis_name="s")

    @functools.partial(
        pl.kernel,
        out_type=jax.ShapeDtypeStruct((NC, N, D), jnp.float32),
        mesh=mesh,
        scratch_types=[
            pltpu.VMEM_SHARED((NA, D), jnp.float32),
            [pltpu.VMEM((NB, WIN), jnp.int32) for _ in range(2)],
            [pltpu.VMEM((NB, WIN), jnp.int32) for _ in range(2)],
            [pltpu.VMEM((WIN, D), jnp.float32) for _ in range(NB)],
            [pltpu.SemaphoreType.DMA for _ in range(NB)],
        ],
    )
    def layer(h_hbm, src_hbm, dst_hbm, zeros_hbm, out_hbm,
              agg, sbuf, dbuf, rows, sems):
        cid = lax.axis_index("c")
        sid = lax.axis_index("s")
        wid = sid * NC + cid

        zbase = sid * RPT
        pltpu.sync_copy(zeros_hbm.at[pl.ds(zbase, RPT)],
                        agg.at[pl.ds(zbase, RPT)])
        if REM:
            @pl.when(sid == NS - 1)
            def _():
                pltpu.sync_copy(zeros_hbm.at[pl.ds(RPT * NS, REM)],
                                agg.at[pl.ds(RPT * NS, REM)])
        pltpu.sync_copy(src_hbm.at[wid, 0], sbuf[0])
        pltpu.sync_copy(dst_hbm.at[wid, 0], dbuf[0])
        plsc.subcore_barrier()

        for b in range(NB):
            pltpu.async_copy(h_hbm.at[sbuf[0].at[b]], rows[b], sems[b])

        def body(g, igrp, ngrp):
            @pl.when(g + 1 < G)
            def _():
                pltpu.sync_copy(src_hbm.at[wid, g + 1], sbuf[ngrp])
                pltpu.sync_copy(dst_hbm.at[wid, g + 1], dbuf[ngrp])
            for b in range(NB):
                pltpu.make_async_copy(h_hbm.at[pl.ds(0, WIN)],
                                      rows[b], sems[b]).wait()
                pltpu.sync_copy(rows[b], agg.at[dbuf[igrp].at[b]],
                                add=True)

                @pl.when(g + 1 < G)
                def _():
                    pltpu.async_copy(h_hbm.at[sbuf[ngrp].at[b]],
                                     rows[b], sems[b])

        def body2(i, _):
            body(2 * i, 0, 1)
            body(2 * i + 1, 1, 0)
            return ()

        assert G % 2 == 0
        lax.fori_loop(0, G // 2, body2, ())

        plsc.subcore_barrier()
        obase = sid * ORPT
        pltpu.sync_copy(agg.at[pl.ds(obase, ORPT)],
                        out_hbm.at[cid, pl.ds(obase, ORPT)])
        if OREM:
            @pl.when(sid == NS - 1)
            def _():
                pltpu.sync_copy(agg.at[pl.ds(ORPT * NS, OREM)],
                                out_hbm.at[cid, pl.ds(ORPT * NS, OREM)])

    return layer


@functools.cache
def _edge_windows(N, E):
    NW = NC * NS
    EC = E // NW
    WPT = -(-(-(-EC // WIN)) // NB) * NB
    PADE = WPT * WIN - EC
    return NW, EC, WPT, PADE


def _pad_edges(src, dst, N, E):
    NW, EC, WPT, PADE = _edge_windows(N, E)
    G = WPT // NB
    j = np.arange(PADE, dtype=np.int32)
    pad_src = jnp.asarray(np.broadcast_to((j * 8) % N, (NW, PADE)))
    pad_dst = jnp.asarray(np.broadcast_to(N + (j % PADROWS), (NW, PADE)))
    srcp = jnp.concatenate([src.reshape(NW, EC), pad_src], axis=1)
    dstp = jnp.concatenate([dst.reshape(NW, EC), pad_dst], axis=1)
    return (srcp.reshape(NW, G, NB, WIN), dstp.reshape(NW, G, NB, WIN))


def _mm_relu_body(x_ref, w_ref, b_ref, o_ref):
    acc = jnp.dot(x_ref[...], w_ref[...], preferred_element_type=jnp.float32)
    o_ref[...] = jnp.maximum(acc + b_ref[...], 0.0)


def _pair_mm_relu_body(p_ref, w_ref, b_ref, o_ref):
    s = p_ref[0] + p_ref[1]
    acc = jnp.dot(s, w_ref[...], preferred_element_type=jnp.float32)
    o_ref[...] = jnp.maximum(acc + b_ref[...], 0.0)


def _mm_relu(x, w, b, blk=2000):
    N, D = x.shape
    assert N % blk == 0
    return pl.pallas_call(
        _mm_relu_body,
        grid=(N // blk,),
        in_specs=[
            pl.BlockSpec((blk, D), lambda i: (i, 0)),
            pl.BlockSpec((D, D), lambda i: (0, 0)),
            pl.BlockSpec((1, D), lambda i: (0, 0)),
        ],
        out_specs=pl.BlockSpec((blk, D), lambda i: (i, 0)),
        out_shape=jax.ShapeDtypeStruct((N, D), jnp.float32),
    )(x, w, b.reshape(1, D))


def _pair_mm_relu(p, w, b, blk=2000):
    _, N, D = p.shape
    assert N % blk == 0
    return pl.pallas_call(
        _pair_mm_relu_body,
        grid=(N // blk,),
        in_specs=[
            pl.BlockSpec((NC, blk, D), lambda i: (0, i, 0)),
            pl.BlockSpec((D, D), lambda i: (0, 0)),
            pl.BlockSpec((1, D), lambda i: (0, 0)),
        ],
        out_specs=pl.BlockSpec((blk, D), lambda i: (i, 0)),
        out_shape=jax.ShapeDtypeStruct((N, D), jnp.float32),
    )(p, w, b.reshape(1, D))


def kernel(x, edge_index, W0, b0, W1, b1, W2, b2):
    N, D = x.shape
    E = edge_index.shape[1]
    srcp, dstp = _pad_edges(edge_index[0], edge_index[1], N, E)
    zeros = jnp.asarray(np.zeros((N + PADROWS, D), np.float32))

    sc_layer = _make_sc_layer(N, E, D)

    h = _mm_relu(x, W0, b0)
    p = sc_layer(h, srcp, dstp, zeros)
    h = _pair_mm_relu(p, W1, b1)
    p = sc_layer(h, srcp, dstp, zeros)
    h = _pair_mm_relu(p, W2, b2)
    return h


# speedup vs baseline: 1.1621x; 1.1621x over previous
import functools

import numpy as np

import jax
import jax.numpy as jnp
from jax import lax
from jax.experimental import pallas as pl
from jax.experimental.pallas import tpu as pltpu
from jax.experimental.pallas import tpu_sc as plsc

NC = 2
NS = 16
WIN = 120


PADROWS = 16
NB = 3


@functools.cache
def _make_sc_layer(N, E, D):
    NW = NC * NS
    assert E % NW == 0
    EC = E // NW
    WPT = -(-(-(-EC // WIN)) // NB) * NB
    NA = N + PADROWS
    G = WPT // NB
    RPT = (NA // NS) // 8 * 8
    REM = NA - RPT * NS
    assert REM % 8 == 0
    ORPT = (N // NS) // 8 * 8
    OREM = N - ORPT * NS

    mesh = plsc.VectorSubcoreMesh(core_axis_name="c", subcore_axis_name="s")

    @functools.partial(
        pl.kernel,
        out_type=jax.ShapeDtypeStruct((NC, N, D), jnp.float32),
        mesh=mesh,
        scratch_types=[
            pltpu.VMEM_SHARED((NA, D), jnp.float32),
            [pltpu.VMEM((NB, WIN), jnp.int32) for _ in range(2)],
            [pltpu.VMEM((NB, WIN), jnp.int32) for _ in range(2)],
            [pltpu.VMEM((WIN, D), jnp.float32) for _ in range(NB)],
            [pltpu.SemaphoreType.DMA for _ in range(NB)],
            [pltpu.SemaphoreType.DMA for _ in range(2)],
        ],
    )
    def layer(h_hbm, src_hbm, dst_hbm, zeros_hbm, out_hbm,
              agg, sbuf, dbuf, rows, sems, isem):
        cid = lax.axis_index("c")
        sid = lax.axis_index("s")
        wid = sid * NC + cid

        zbase = sid * RPT
        pltpu.sync_copy(zeros_hbm.at[pl.ds(zbase, RPT)],
                        agg.at[pl.ds(zbase, RPT)])
        if REM:
            @pl.when(sid == NS - 1)
            def _():
                pltpu.sync_copy(zeros_hbm.at[pl.ds(RPT * NS, REM)],
                                agg.at[pl.ds(RPT * NS, REM)])
        pltpu.sync_copy(src_hbm.at[wid, 0], sbuf[0])
        pltpu.sync_copy(dst_hbm.at[wid, 0], dbuf[0])
        plsc.subcore_barrier()

        for b in range(NB):
            pltpu.async_copy(h_hbm.at[sbuf[0].at[b]], rows[b], sems[b])

        def body(g, igrp, ngrp):
            @pl.when(g + 1 < G)
            def _():
                pltpu.async_copy(src_hbm.at[wid, g + 1], sbuf[ngrp],
                                 isem[ngrp])
                pltpu.async_copy(dst_hbm.at[wid, g + 1], dbuf[ngrp],
                                 isem[ngrp])
            for b in range(NB):
                pltpu.make_async_copy(h_hbm.at[pl.ds(0, WIN)],
                                      rows[b], sems[b]).wait()
                pltpu.sync_copy(rows[b], agg.at[dbuf[igrp].at[b]],
                                add=True)

                @pl.when(g + 1 < G)
                def _():
                    if b == 0:
                        pltpu.make_async_copy(src_hbm.at[wid, 0],
                                              sbuf[ngrp], isem[ngrp]).wait()
                        pltpu.make_async_copy(src_hbm.at[wid, 0],
                                              dbuf[ngrp], isem[ngrp]).wait()
                    pltpu.async_copy(h_hbm.at[sbuf[ngrp].at[b]],
                                     rows[b], sems[b])

        def body2(i, _):
            body(2 * i, 0, 1)
            body(2 * i + 1, 1, 0)
            return ()

        assert G % 2 == 0
        lax.fori_loop(0, G // 2, body2, ())

        plsc.subcore_barrier()
        obase = sid * ORPT
        pltpu.sync_copy(agg.at[pl.ds(obase, ORPT)],
                        out_hbm.at[cid, pl.ds(obase, ORPT)])
        if OREM:
            @pl.when(sid == NS - 1)
            def _():
                pltpu.sync_copy(agg.at[pl.ds(ORPT * NS, OREM)],
                                out_hbm.at[cid, pl.ds(ORPT * NS, OREM)])

    return layer


@functools.cache
def _edge_windows(N, E):
    NW = NC * NS
    EC = E // NW
    WPT = -(-(-(-EC // WIN)) // NB) * NB
    PADE = WPT * WIN - EC
    return NW, EC, WPT, PADE


def _pad_edges(src, dst, N, E):
    NW, EC, WPT, PADE = _edge_windows(N, E)
    G = WPT // NB
    j = np.arange(PADE, dtype=np.int32)
    pad_src = jnp.asarray(np.broadcast_to((j * 8) % N, (NW, PADE)))
    pad_dst = jnp.asarray(np.broadcast_to(N + (j % PADROWS), (NW, PADE)))
    srcp = jnp.concatenate([src.reshape(NW, EC), pad_src], axis=1)
    dstp = jnp.concatenate([dst.reshape(NW, EC), pad_dst], axis=1)
    return (srcp.reshape(NW, G, NB, WIN), dstp.reshape(NW, G, NB, WIN))


def _mm_relu_body(x_ref, w_ref, b_ref, o_ref):
    acc = jnp.dot(x_ref[...], w_ref[...], preferred_element_type=jnp.float32)
    o_ref[...] = jnp.maximum(acc + b_ref[...], 0.0)


def _pair_mm_relu_body(p_ref, w_ref, b_ref, o_ref):
    s = p_ref[0] + p_ref[1]
    acc = jnp.dot(s, w_ref[...], preferred_element_type=jnp.float32)
    o_ref[...] = jnp.maximum(acc + b_ref[...], 0.0)


def _mm_relu(x, w, b, blk=2000):
    N, D = x.shape
    assert N % blk == 0
    return pl.pallas_call(
        _mm_relu_body,
        grid=(N // blk,),
        in_specs=[
            pl.BlockSpec((blk, D), lambda i: (i, 0)),
            pl.BlockSpec((D, D), lambda i: (0, 0)),
            pl.BlockSpec((1, D), lambda i: (0, 0)),
        ],
        out_specs=pl.BlockSpec((blk, D), lambda i: (i, 0)),
        out_shape=jax.ShapeDtypeStruct((N, D), jnp.float32),
    )(x, w, b.reshape(1, D))


def _pair_mm_relu(p, w, b, blk=2000):
    _, N, D = p.shape
    assert N % blk == 0
    return pl.pallas_call(
        _pair_mm_relu_body,
        grid=(N // blk,),
        in_specs=[
            pl.BlockSpec((NC, blk, D), lambda i: (0, i, 0)),
            pl.BlockSpec((D, D), lambda i: (0, 0)),
            pl.BlockSpec((1, D), lambda i: (0, 0)),
        ],
        out_specs=pl.BlockSpec((blk, D), lambda i: (i, 0)),
        out_shape=jax.ShapeDtypeStruct((N, D), jnp.float32),
    )(p, w, b.reshape(1, D))


def kernel(x, edge_index, W0, b0, W1, b1, W2, b2):
    N, D = x.shape
    E = edge_index.shape[1]
    srcp, dstp = _pad_edges(edge_index[0], edge_index[1], N, E)
    zeros = jnp.asarray(np.zeros((N + PADROWS, D), np.float32))

    sc_layer = _make_sc_layer(N, E, D)

    h = _mm_relu(x, W0, b0)
    p = sc_layer(h, srcp, dstp, zeros)
    h = _pair_mm_relu(p, W1, b1)
    p = sc_layer(h, srcp, dstp, zeros)
    h = _pair_mm_relu(p, W2, b2)
    return h


# speedup vs baseline: 1.1799x; 1.0153x over previous
import functools

import numpy as np

import jax
import jax.numpy as jnp
from jax import lax
from jax.experimental import pallas as pl
from jax.experimental.pallas import tpu as pltpu
from jax.experimental.pallas import tpu_sc as plsc

NC = 2
NS = 16
WIN = 128
NB = 3


@functools.cache
def _make_sc_layer(N, E, D):
    NW = NC * NS
    assert E % WIN == 0
    TW = E // WIN
    FW = TW // NW
    EXTRA = TW - FW * NW
    G = FW // NB
    assert FW % NB == 0 and G % 2 == 0 and EXTRA < NW
    RPT = (N // NS) // 8 * 8
    REM = N - RPT * NS
    assert REM % 8 == 0

    mesh = plsc.VectorSubcoreMesh(core_axis_name="c", subcore_axis_name="s")

    @functools.partial(
        pl.kernel,
        out_type=jax.ShapeDtypeStruct((NC, N, D), jnp.float32),
        mesh=mesh,
        scratch_types=[
            pltpu.VMEM_SHARED((N, D), jnp.float32),
            [pltpu.VMEM((NB * WIN,), jnp.int32) for _ in range(2)],
            [pltpu.VMEM((NB, WIN), jnp.int32) for _ in range(2)],
            [pltpu.VMEM((WIN, D), jnp.float32) for _ in range(NB)],
            [pltpu.SemaphoreType.DMA for _ in range(NB)],
            [pltpu.SemaphoreType.DMA for _ in range(2)],
        ],
    )
    def layer(h_hbm, src_hbm, dst_hbm, zeros_hbm, out_hbm,
              agg, sbuf, dbuf, rows, sems, isem):
        cid = lax.axis_index("c")
        sid = lax.axis_index("s")
        wid = sid * NC + cid
        wbase = wid * FW

        zbase = sid * RPT
        pltpu.sync_copy(zeros_hbm.at[pl.ds(zbase, RPT)],
                        agg.at[pl.ds(zbase, RPT)])
        if REM:
            @pl.when(sid == NS - 1)
            def _():
                pltpu.sync_copy(zeros_hbm.at[pl.ds(RPT * NS, REM)],
                                agg.at[pl.ds(RPT * NS, REM)])
        pltpu.sync_copy(src_hbm.at[pl.ds(wbase * WIN, NB * WIN)], sbuf[0])
        for b in range(NB):
            pltpu.sync_copy(dst_hbm.at[pl.ds((wbase + b) * WIN, WIN)],
                            dbuf[0].at[b])
        plsc.subcore_barrier()

        for b in range(NB):
            pltpu.async_copy(h_hbm.at[sbuf[0].at[pl.ds(b * WIN, WIN)]],
                             rows[b], sems[b])

        def body(g, igrp, ngrp):
            @pl.when(g + 1 < G)
            def _():
                nb_ = (wbase + (g + 1) * NB) * WIN
                pltpu.async_copy(src_hbm.at[pl.ds(nb_, NB * WIN)],
                                 sbuf[ngrp], isem[ngrp])
                for b in range(NB):
                    pltpu.async_copy(dst_hbm.at[pl.ds(nb_ + b * WIN, WIN)],
                                     dbuf[ngrp].at[b], isem[ngrp])
            for b in range(NB):
                pltpu.make_async_copy(h_hbm.at[pl.ds(0, WIN)],
                                      rows[b], sems[b]).wait()
                pltpu.sync_copy(rows[b], agg.at[dbuf[igrp].at[b]],
                                add=True)

                @pl.when(g + 1 < G)
                def _():
                    if b == 0:
                        pltpu.make_async_copy(
                            src_hbm.at[pl.ds(0, NB * WIN)],
                            sbuf[ngrp], isem[ngrp]).wait()
                        for bb in range(NB):
                            pltpu.make_async_copy(
                                src_hbm.at[pl.ds(0, WIN)],
                                dbuf[ngrp].at[bb], isem[ngrp]).wait()
                    pltpu.async_copy(
                        h_hbm.at[sbuf[ngrp].at[pl.ds(b * WIN, WIN)]],
                        rows[b], sems[b])

        def body2(i, _):
            body(2 * i, 0, 1)
            body(2 * i + 1, 1, 0)
            return ()

        lax.fori_loop(0, G // 2, body2, ())

        if EXTRA:
            @pl.when(wid < EXTRA)
            def _():
                xoff = (FW * NW + wid) * WIN
                pltpu.sync_copy(src_hbm.at[pl.ds(xoff, WIN)],
                                sbuf[0].at[pl.ds(0, WIN)])
                pltpu.sync_copy(dst_hbm.at[pl.ds(xoff, WIN)],
                                dbuf[0].at[0])
                pltpu.async_copy(h_hbm.at[sbuf[0].at[pl.ds(0, WIN)]],
                                 rows[0], sems[0]).wait()
                pltpu.sync_copy(rows[0], agg.at[dbuf[0].at[0]], add=True)

        plsc.subcore_barrier()
        pltpu.sync_copy(agg.at[pl.ds(zbase, RPT)],
                        out_hbm.at[cid, pl.ds(zbase, RPT)])
        if REM:
            @pl.when(sid == NS - 1)
            def _():
                pltpu.sync_copy(agg.at[pl.ds(RPT * NS, REM)],
                                out_hbm.at[cid, pl.ds(RPT * NS, REM)])

    return layer


def _mm_relu_body(x_ref, w_ref, b_ref, o_ref):
    acc = jnp.dot(x_ref[...], w_ref[...], preferred_element_type=jnp.float32)
    o_ref[...] = jnp.maximum(acc + b_ref[...], 0.0)


def _pair_mm_relu_body(p_ref, w_ref, b_ref, o_ref):
    s = p_ref[0] + p_ref[1]
    acc = jnp.dot(s, w_ref[...], preferred_element_type=jnp.float32)
    o_ref[...] = jnp.maximum(acc + b_ref[...], 0.0)


def _mm_relu(x, w, b, blk=2000):
    N, D = x.shape
    assert N % blk == 0
    return pl.pallas_call(
        _mm_relu_body,
        grid=(N // blk,),
        in_specs=[
            pl.BlockSpec((blk, D), lambda i: (i, 0)),
            pl.BlockSpec((D, D), lambda i: (0, 0)),
            pl.BlockSpec((1, D), lambda i: (0, 0)),
        ],
        out_specs=pl.BlockSpec((blk, D), lambda i: (i, 0)),
        out_shape=jax.ShapeDtypeStruct((N, D), jnp.float32),
    )(x, w, b.reshape(1, D))


def _pair_mm_relu(p, w, b, blk=2000):
    _, N, D = p.shape
    assert N % blk == 0
    return pl.pallas_call(
        _pair_mm_relu_body,
        grid=(N // blk,),
        in_specs=[
            pl.BlockSpec((NC, blk, D), lambda i: (0, i, 0)),
            pl.BlockSpec((D, D), lambda i: (0, 0)),
            pl.BlockSpec((1, D), lambda i: (0, 0)),
        ],
        out_specs=pl.BlockSpec((blk, D), lambda i: (i, 0)),
        out_shape=jax.ShapeDtypeStruct((N, D), jnp.float32),
    )(p, w, b.reshape(1, D))


def kernel(x, edge_index, W0, b0, W1, b1, W2, b2):
    N, D = x.shape
    E = edge_index.shape[1]
    src2 = edge_index[0]
    dst2 = edge_index[1]
    zeros = jnp.asarray(np.zeros((N, D), np.float32))

    sc_layer = _make_sc_layer(N, E, D)

    h = _mm_relu(x, W0, b0)
    p = sc_layer(h, src2, dst2, zeros)
    h = _pair_mm_relu(p, W1, b1)
    p = sc_layer(h, src2, dst2, zeros)
    h = _pair_mm_relu(p, W2, b2)
    return h


# speedup vs baseline: 1.2344x; 1.0461x over previous
import functools

import numpy as np

import jax
import jax.numpy as jnp
from jax import lax
from jax.experimental import pallas as pl
from jax.experimental.pallas import tpu as pltpu
from jax.experimental.pallas import tpu_sc as plsc

NC = 2
NS = 16
WIN = 128
NB = 3


@functools.cache
def _make_sc_layer(N, E, D):
    NW = NC * NS
    assert E % WIN == 0
    TW = E // WIN
    FW = TW // NW
    EXTRA = TW - FW * NW
    G = FW // NB
    assert FW % NB == 0 and G % 2 == 0 and EXTRA < NW
    RPT = (N // NS) // 8 * 8
    REM = N - RPT * NS
    assert REM % 8 == 0

    mesh = plsc.VectorSubcoreMesh(core_axis_name="c", subcore_axis_name="s")

    @functools.partial(
        pl.kernel,
        out_type=jax.ShapeDtypeStruct((NC, N, D), jnp.float32),
        mesh=mesh,
        scratch_types=[
            pltpu.VMEM_SHARED((N, D), jnp.float32),
            [pltpu.VMEM((NB * WIN,), jnp.int32) for _ in range(2)],
            [pltpu.VMEM((NB, WIN), jnp.int32) for _ in range(2)],
            [pltpu.VMEM((WIN, D), jnp.float32) for _ in range(NB)],
            [pltpu.SemaphoreType.DMA for _ in range(NB)],
            [pltpu.SemaphoreType.DMA for _ in range(2)],
        ],
    )
    def layer(h_hbm, src_hbm, dst_hbm, zeros_hbm, out_hbm,
              agg, sbuf, dbuf, rows, sems, isem):
        cid = lax.axis_index("c")
        sid = lax.axis_index("s")
        wid = sid * NC + cid
        wbase = wid * FW

        zbase = sid * RPT
        pltpu.sync_copy(zeros_hbm.at[pl.ds(zbase, RPT)],
                        agg.at[pl.ds(zbase, RPT)])
        if REM:
            @pl.when(sid == NS - 1)
            def _():
                pltpu.sync_copy(zeros_hbm.at[pl.ds(RPT * NS, REM)],
                                agg.at[pl.ds(RPT * NS, REM)])
        pltpu.sync_copy(src_hbm.at[pl.ds(wbase * WIN, NB * WIN)], sbuf[0])
        for b in range(NB):
            pltpu.sync_copy(dst_hbm.at[pl.ds((wbase + b) * WIN, WIN)],
                            dbuf[0].at[b])
        plsc.subcore_barrier()

        for b in range(NB):
            pltpu.async_copy(h_hbm.at[sbuf[0].at[pl.ds(b * WIN, WIN)]],
                             rows[b], sems[b])

        def body(g, igrp, ngrp):
            @pl.when(g + 1 < G)
            def _():
                nb_ = (wbase + (g + 1) * NB) * WIN
                pltpu.async_copy(src_hbm.at[pl.ds(nb_, NB * WIN)],
                                 sbuf[ngrp], isem[ngrp])
                for b in range(NB):
                    pltpu.async_copy(dst_hbm.at[pl.ds(nb_ + b * WIN, WIN)],
                                     dbuf[ngrp].at[b], isem[ngrp])
            for b in range(NB):
                pltpu.make_async_copy(h_hbm.at[pl.ds(0, WIN)],
                                      rows[b], sems[b]).wait()
                pltpu.sync_copy(rows[b], agg.at[dbuf[igrp].at[b]],
                                add=True)

                @pl.when(g + 1 < G)
                def _():
                    if b == 0:
                        pltpu.make_async_copy(
                            src_hbm.at[pl.ds(0, NB * WIN)],
                            sbuf[ngrp], isem[ngrp]).wait()
                        for bb in range(NB):
                            pltpu.make_async_copy(
                                src_hbm.at[pl.ds(0, WIN)],
                                dbuf[ngrp].at[bb], isem[ngrp]).wait()
                    pltpu.async_copy(
                        h_hbm.at[sbuf[ngrp].at[pl.ds(b * WIN, WIN)]],
                        rows[b], sems[b])

        def body2(i, _):
            body(2 * i, 0, 1)
            body(2 * i + 1, 1, 0)
            return ()

        lax.fori_loop(0, G // 2, body2, ())

        if EXTRA:
            @pl.when(wid < EXTRA)
            def _():
                xoff = (FW * NW + wid) * WIN
                pltpu.sync_copy(src_hbm.at[pl.ds(xoff, WIN)],
                                sbuf[0].at[pl.ds(0, WIN)])
                pltpu.sync_copy(dst_hbm.at[pl.ds(xoff, WIN)],
                                dbuf[0].at[0])
                pltpu.async_copy(h_hbm.at[sbuf[0].at[pl.ds(0, WIN)]],
                                 rows[0], sems[0]).wait()
                pltpu.sync_copy(rows[0], agg.at[dbuf[0].at[0]], add=True)

        plsc.subcore_barrier()
        pltpu.sync_copy(agg.at[pl.ds(zbase, RPT)],
                        out_hbm.at[cid, pl.ds(zbase, RPT)])
        if REM:
            @pl.when(sid == NS - 1)
            def _():
                pltpu.sync_copy(agg.at[pl.ds(RPT * NS, REM)],
                                out_hbm.at[cid, pl.ds(RPT * NS, REM)])

    return layer


def _split_edges_body(e_ref, s_ref, d_ref):
    s_ref[...] = e_ref[0]
    d_ref[...] = e_ref[1]


def _split_edges(edge_index):
    _, E = edge_index.shape
    blk = E
    return pl.pallas_call(
        _split_edges_body,
        grid=(1,),
        in_specs=[pl.BlockSpec((2, blk), lambda i: (0, 0))],
        out_specs=[pl.BlockSpec((blk,), lambda i: (0,)),
                   pl.BlockSpec((blk,), lambda i: (0,))],
        out_shape=[jax.ShapeDtypeStruct((E,), jnp.int32),
                   jax.ShapeDtypeStruct((E,), jnp.int32)],
    )(edge_index)


def _mm_relu_body(x_ref, w_ref, b_ref, o_ref):
    acc = jnp.dot(x_ref[...], w_ref[...], preferred_element_type=jnp.float32)
    o_ref[...] = jnp.maximum(acc + b_ref[...], 0.0)


def _pair_mm_relu_body(p_ref, w_ref, b_ref, o_ref):
    s = p_ref[0] + p_ref[1]
    acc = jnp.dot(s, w_ref[...], preferred_element_type=jnp.float32)
    o_ref[...] = jnp.maximum(acc + b_ref[...], 0.0)


def _mm_relu(x, w, b, blk=2000):
    N, D = x.shape
    assert N % blk == 0
    return pl.pallas_call(
        _mm_relu_body,
        grid=(N // blk,),
        in_specs=[
            pl.BlockSpec((blk, D), lambda i: (i, 0)),
            pl.BlockSpec((D, D), lambda i: (0, 0)),
            pl.BlockSpec((1, D), lambda i: (0, 0)),
        ],
        out_specs=pl.BlockSpec((blk, D), lambda i: (i, 0)),
        out_shape=jax.ShapeDtypeStruct((N, D), jnp.float32),
    )(x, w, b.reshape(1, D))


def _pair_mm_relu(p, w, b, blk=2000):
    _, N, D = p.shape
    assert N % blk == 0
    return pl.pallas_call(
        _pair_mm_relu_body,
        grid=(N // blk,),
        in_specs=[
            pl.BlockSpec((NC, blk, D), lambda i: (0, i, 0)),
            pl.BlockSpec((D, D), lambda i: (0, 0)),
            pl.BlockSpec((1, D), lambda i: (0, 0)),
        ],
        out_specs=pl.BlockSpec((blk, D), lambda i: (i, 0)),
        out_shape=jax.ShapeDtypeStruct((N, D), jnp.float32),
    )(p, w, b.reshape(1, D))


def kernel(x, edge_index, W0, b0, W1, b1, W2, b2):
    N, D = x.shape
    E = edge_index.shape[1]
    src2, dst2 = _split_edges(edge_index)
    zeros = jnp.asarray(np.zeros((N, D), np.float32))

    sc_layer = _make_sc_layer(N, E, D)

    h = _mm_relu(x, W0, b0)
    p = sc_layer(h, src2, dst2, zeros)
    h = _pair_mm_relu(p, W1, b1)
    p = sc_layer(h, src2, dst2, zeros)
    h = _pair_mm_relu(p, W2, b2)
    return h
